# baseline (device time: 250948 ns/iter reference)
import os
SKIP_COMM = bool(int(os.environ.get('SKIP_COMM', '0')))
FULL_WAIT = bool(int(os.environ.get('FULL_WAIT', '0')))
import jax
import jax.numpy as jnp
from jax import lax
from jax.experimental import pallas as pl
from jax.experimental.pallas import tpu as pltpu

N_DEV = 32
B, Sq, Hq, Hkv, Dh = 4, 256, 8, 2, 128
G = Hq // Hkv
D = Hq * Dh
R = B * Sq
CHUNK = R // N_DEV
SCALE = 0.08838834764831843
NR = N_DEV // 2
NL = N_DEV // 2 - 1
BF = jnp.bfloat16


def kernel(x, Wq, Wo, K_ext, V_ext):
    x_flat = x.reshape(R, D)

    def body(x_ref, wq_ref, wo_ref, k_ref, v_ref, out_ref,
             acc_o, lsum, lr_o, lr_st, ll_o, ll_st,
             sr_o, sr_st, sl_o, sl_st, sag_r, sag_l,
             r_o_sems, r_st_sems, l_o_sems, l_st_sems,
             agr_sems, agl_sems):
        my = lax.axis_index("i")
        left = lax.rem(my - 1 + N_DEV, N_DEV)
        right = lax.rem(my + 1, N_DEV)

        barrier = pltpu.get_barrier_semaphore()
        for nbr in (left, right):
            pl.semaphore_signal(
                barrier, inc=1,
                device_id=(nbr,), device_id_type=pl.DeviceIdType.MESH,
            )
        pl.semaphore_wait(barrier, 2)

        def batch_step(b, carry):
            rows = pl.ds(b * Sq, Sq)
            xb = x_ref[rows, :].astype(BF)
            for h in range(Hq):
                g = h // G
                q = jnp.dot(
                    xb, wq_ref[:, h * Dh:(h + 1) * Dh].astype(BF),
                    preferred_element_type=jnp.float32,
                )
                k = k_ref[b, :, g, :].astype(BF)
                v = v_ref[b, :, g, :].astype(BF)
                s = lax.dot_general(
                    q.astype(BF), k, (((1,), (1,)), ((), ())),
                    preferred_element_type=jnp.float32,
                ) * SCALE
                p = jnp.exp(s)
                l_loc = jnp.sum(p, axis=1, keepdims=True)
                o = jnp.dot(p.astype(BF), v,
                            preferred_element_type=jnp.float32)
                acc_o[rows, h * Dh:(h + 1) * Dh] = o
                lsum[rows, h:h + 1] = l_loc
            return carry

        lax.fori_loop(0, B, batch_step, 0)

        def send_pair(chunk, o_land, st_land, slot, o_ssem, st_ssem,
                      o_rsems, st_rsems, dev):
            rows = pl.ds(chunk * CHUNK, CHUNK)
            rd_o = pltpu.make_async_remote_copy(
                src_ref=acc_o.at[rows, :],
                dst_ref=o_land.at[slot],
                send_sem=o_ssem, recv_sem=o_rsems.at[slot],
                device_id=(dev,), device_id_type=pl.DeviceIdType.MESH,
            )
            rd_st = pltpu.make_async_remote_copy(
                src_ref=lsum.at[rows, :],
                dst_ref=st_land.at[slot],
                send_sem=st_ssem, recv_sem=st_rsems.at[slot],
                device_id=(dev,), device_id_type=pl.DeviceIdType.MESH,
            )
            rd_o.start()
            rd_st.start()
            return rd_o, rd_st

        prev = []
        for s in range(0 if SKIP_COMM else NR):
            for rd in prev:
                rd.wait_send()
            sc_r = lax.rem(my + NR - s, N_DEV)
            pend = list(send_pair(sc_r, lr_o, lr_st, s, sr_o, sr_st,
                                  r_o_sems, r_st_sems, right))
            if s < NL:
                sc_l = lax.rem(my - NL + s + N_DEV, N_DEV)
                pend += send_pair(sc_l, ll_o, ll_st, s, sl_o, sl_st,
                                  l_o_sems, l_st_sems, left)
            for rd in pend:
                (rd.wait if FULL_WAIT else rd.wait_recv)()
            prev = [] if FULL_WAIT else pend
            rc = lax.rem(my + NR - 1 - s + N_DEV, N_DEV)
            rows = pl.ds(rc * CHUNK, CHUNK)
            acc_o[rows, :] = acc_o[rows, :] + lr_o[s]
            lsum[rows, :] = lsum[rows, :] + lr_st[s]
            if s < NL:
                rc = lax.rem(my - NL + 1 + s + N_DEV, N_DEV)
                rows = pl.ds(rc * CHUNK, CHUNK)
                acc_o[rows, :] = acc_o[rows, :] + ll_o[s]
                lsum[rows, :] = lsum[rows, :] + ll_st[s]
        for rd in prev:
            rd.wait_send()

        orows = pl.ds(my * CHUNK, CHUNK)
        linv = 1.0 / lsum[orows, :]
        och = acc_o[orows, :]
        norm = jnp.concatenate(
            [och[:, h * Dh:(h + 1) * Dh] * linv[:, h:h + 1]
             for h in range(Hq)],
            axis=1,
        )
        out_ref[orows, :] = jnp.dot(
            norm.astype(BF), wo_ref[:, :].astype(BF),
            preferred_element_type=jnp.float32,
        )

        prev = []
        for a in range(0 if SKIP_COMM else NR):
            for rd in prev:
                rd.wait_send()
            pend = []
            sc_r = lax.rem(my - a + N_DEV, N_DEV)
            srows = pl.ds(sc_r * CHUNK, CHUNK)
            rd = pltpu.make_async_remote_copy(
                src_ref=out_ref.at[srows, :],
                dst_ref=out_ref.at[srows, :],
                send_sem=sag_r, recv_sem=agr_sems.at[a],
                device_id=(right,), device_id_type=pl.DeviceIdType.MESH,
            )
            rd.start()
            pend.append(rd)
            if a < NL:
                sc_l = lax.rem(my + a, N_DEV)
                srows = pl.ds(sc_l * CHUNK, CHUNK)
                rd = pltpu.make_async_remote_copy(
                    src_ref=out_ref.at[srows, :],
                    dst_ref=out_ref.at[srows, :],
                    send_sem=sag_l, recv_sem=agl_sems.at[a],
                    device_id=(left,), device_id_type=pl.DeviceIdType.MESH,
                )
                rd.start()
                pend.append(rd)
            for rd in pend:
                (rd.wait if FULL_WAIT else rd.wait_recv)()
            prev = [] if FULL_WAIT else pend
        for rd in prev:
            rd.wait_send()

    out = pl.pallas_call(
        body,
        out_shape=jax.ShapeDtypeStruct((R, D), jnp.float32),
        in_specs=[pl.BlockSpec(memory_space=pltpu.VMEM)] * 5,
        out_specs=pl.BlockSpec(memory_space=pltpu.VMEM),
        scratch_shapes=[
            pltpu.VMEM((R, D), jnp.float32),
            pltpu.VMEM((R, Hq), jnp.float32),
            pltpu.VMEM((NR, CHUNK, D), jnp.float32),
            pltpu.VMEM((NR, CHUNK, Hq), jnp.float32),
            pltpu.VMEM((NL, CHUNK, D), jnp.float32),
            pltpu.VMEM((NL, CHUNK, Hq), jnp.float32),
            pltpu.SemaphoreType.DMA,
            pltpu.SemaphoreType.DMA,
            pltpu.SemaphoreType.DMA,
            pltpu.SemaphoreType.DMA,
            pltpu.SemaphoreType.DMA,
            pltpu.SemaphoreType.DMA,
            pltpu.SemaphoreType.DMA((NR,)),
            pltpu.SemaphoreType.DMA((NR,)),
            pltpu.SemaphoreType.DMA((NL,)),
            pltpu.SemaphoreType.DMA((NL,)),
            pltpu.SemaphoreType.DMA((NR,)),
            pltpu.SemaphoreType.DMA((NL,)),
        ],
        compiler_params=pltpu.CompilerParams(
            collective_id=0,
            vmem_limit_bytes=100 * 1024 * 1024,
        ),
    )(x_flat, Wq, Wo, K_ext, V_ext)
    return out.reshape(B, Sq, D)


# device time: 44077 ns/iter; 5.6934x vs baseline; 5.6934x over previous
import os
SKIP_COMM = bool(int(os.environ.get('SKIP_COMM', '0')))
import jax
import jax.numpy as jnp
from jax import lax
from jax.experimental import pallas as pl
from jax.experimental.pallas import tpu as pltpu

N_DEV = 32
B, Sq, Hq, Hkv, Dh = 4, 256, 8, 2, 128
G = Hq // Hkv
D = Hq * Dh
W = D + 128
R = B * Sq
CHUNK = R // N_DEV
SCALE = 0.08838834764831843
NR = N_DEV // 2
NL = N_DEV // 2 - 1
BF = jnp.bfloat16


def kernel(x, Wq, Wo, K_ext, V_ext):
    x_flat = x.reshape(R, D)

    def body(x_ref, wq_ref, wo_ref, k_ref, v_ref, out_ref,
             acc, lr_land, ll_land,
             sr_sem, sl_sem, sag_r, sag_l,
             r_sems, l_sems, agr_sems, agl_sems):
        my = lax.axis_index("i")
        left = lax.rem(my - 1 + N_DEV, N_DEV)
        right = lax.rem(my + 1, N_DEV)

        barrier = pltpu.get_barrier_semaphore()
        for nbr in (left, right):
            pl.semaphore_signal(
                barrier, inc=1,
                device_id=(nbr,), device_id_type=pl.DeviceIdType.MESH,
            )
        pl.semaphore_wait(barrier, 2)

        out_ref[:, :] = jnp.dot(
            x_ref[:, :].astype(BF), wq_ref[:, :].astype(BF),
            preferred_element_type=jnp.float32,
        )
        for b in range(B):
            rows = slice(b * Sq, (b + 1) * Sq)
            for g in range(Hkv):
                k16 = k_ref[b, :, g, :].astype(BF)
                v16 = v_ref[b, :, g, :].astype(BF)
                for h in range(g * G, (g + 1) * G):
                    q16 = out_ref[rows, h * Dh:(h + 1) * Dh].astype(BF)
                    s = lax.dot_general(
                        q16, k16, (((1,), (1,)), ((), ())),
                        preferred_element_type=jnp.float32,
                    ) * SCALE
                    p = jnp.exp(s)
                    l_loc = jnp.sum(p, axis=1, keepdims=True)
                    o = jnp.dot(p.astype(BF), v16,
                                preferred_element_type=jnp.float32)
                    acc[rows, h * Dh:(h + 1) * Dh] = o
                    acc[rows, D + h:D + h + 1] = l_loc

        def send_chunk(chunk, land, slot, ssem, rsems, dev):
            rows = pl.ds(chunk * CHUNK, CHUNK)
            rd = pltpu.make_async_remote_copy(
                src_ref=acc.at[rows, :],
                dst_ref=land.at[slot],
                send_sem=ssem, recv_sem=rsems.at[slot],
                device_id=(dev,), device_id_type=pl.DeviceIdType.MESH,
            )
            rd.start()
            return rd

        prev = []
        for s in range(0 if SKIP_COMM else NR):
            for rd in prev:
                rd.wait_send()
            sc_r = lax.rem(my + NR - s, N_DEV)
            pend = [send_chunk(sc_r, lr_land, s, sr_sem, r_sems, right)]
            if s < NL:
                sc_l = lax.rem(my - NL + s + N_DEV, N_DEV)
                pend.append(send_chunk(sc_l, ll_land, s, sl_sem, l_sems,
                                       left))
            for rd in pend:
                rd.wait_recv()
            prev = pend
            rc = lax.rem(my + NR - 1 - s + N_DEV, N_DEV)
            rows = pl.ds(rc * CHUNK, CHUNK)
            acc[rows, :] = acc[rows, :] + lr_land[s]
            if s < NL:
                rc = lax.rem(my - NL + 1 + s + N_DEV, N_DEV)
                rows = pl.ds(rc * CHUNK, CHUNK)
                acc[rows, :] = acc[rows, :] + ll_land[s]
        for rd in prev:
            rd.wait_send()

        orows = pl.ds(my * CHUNK, CHUNK)
        linv = 1.0 / acc[orows, D:D + Hq]
        och = acc[orows, 0:D]
        norm = jnp.concatenate(
            [och[:, h * Dh:(h + 1) * Dh] * linv[:, h:h + 1]
             for h in range(Hq)],
            axis=1,
        )
        out_ref[orows, :] = jnp.dot(
            norm.astype(BF), wo_ref[:, :].astype(BF),
            preferred_element_type=jnp.float32,
        )

        prev = []
        for a in range(0 if SKIP_COMM else NR):
            for rd in prev:
                rd.wait_send()
            pend = []
            sc_r = lax.rem(my - a + N_DEV, N_DEV)
            srows = pl.ds(sc_r * CHUNK, CHUNK)
            rd = pltpu.make_async_remote_copy(
                src_ref=out_ref.at[srows, :],
                dst_ref=out_ref.at[srows, :],
                send_sem=sag_r, recv_sem=agr_sems.at[a],
                device_id=(right,), device_id_type=pl.DeviceIdType.MESH,
            )
            rd.start()
            pend.append(rd)
            if a < NL:
                sc_l = lax.rem(my + a, N_DEV)
                srows = pl.ds(sc_l * CHUNK, CHUNK)
                rd = pltpu.make_async_remote_copy(
                    src_ref=out_ref.at[srows, :],
                    dst_ref=out_ref.at[srows, :],
                    send_sem=sag_l, recv_sem=agl_sems.at[a],
                    device_id=(left,), device_id_type=pl.DeviceIdType.MESH,
                )
                rd.start()
                pend.append(rd)
            for rd in pend:
                rd.wait_recv()
            prev = pend
        for rd in prev:
            rd.wait_send()

    out = pl.pallas_call(
        body,
        out_shape=jax.ShapeDtypeStruct((R, D), jnp.float32),
        in_specs=[pl.BlockSpec(memory_space=pltpu.VMEM)] * 5,
        out_specs=pl.BlockSpec(memory_space=pltpu.VMEM),
        scratch_shapes=[
            pltpu.VMEM((R, W), jnp.float32),
            pltpu.VMEM((NR, CHUNK, W), jnp.float32),
            pltpu.VMEM((NL, CHUNK, W), jnp.float32),
            pltpu.SemaphoreType.DMA,
            pltpu.SemaphoreType.DMA,
            pltpu.SemaphoreType.DMA,
            pltpu.SemaphoreType.DMA,
            pltpu.SemaphoreType.DMA((NR,)),
            pltpu.SemaphoreType.DMA((NL,)),
            pltpu.SemaphoreType.DMA((NR,)),
            pltpu.SemaphoreType.DMA((NL,)),
        ],
        compiler_params=pltpu.CompilerParams(
            collective_id=0,
            vmem_limit_bytes=100 * 1024 * 1024,
        ),
    )(x_flat, Wq, Wo, K_ext, V_ext)
    return out.reshape(B, Sq, D)
